# Initial kernel scaffold; baseline (speedup 1.0000x reference)
import jax
import jax.numpy as jnp
from jax.experimental import pallas as pl


def kernel(H, edge_index, edge_attr, W1, b1, W2, b2, gamma, beta):
    raise NotImplementedError("write your pallas kernel here")


# speedup vs baseline: 3.1268x; 3.1268x over previous
import functools

import jax
import jax.numpy as jnp
from jax import lax
from jax.experimental import pallas as pl
from jax.experimental.pallas import tpu as pltpu
from jax.experimental.pallas import tpu_sc as plsc

N = 10000
E = 320000
D = 128
DE = 16

NC = 2
NS = 16
NW = NC * NS
EPW = E // NW
BATCH = 80
NB = EPW // BATCH
NP = 10240
RPS = NP // NS

_sc_mesh = plsc.VectorSubcoreMesh(core_axis_name="c", subcore_axis_name="s")


def _gather_body(table_hbm, idx_hbm, out_hbm, idx_v, rows_v, sem):
    c = lax.axis_index("c")
    s = lax.axis_index("s")
    base = (s * NC + c) * EPW

    def body(i, carry):
        off = base + i * BATCH
        pltpu.sync_copy(idx_hbm.at[pl.ds(off, BATCH)], idx_v)
        pltpu.async_copy(table_hbm.at[idx_v], rows_v, sem).wait()
        pltpu.sync_copy(rows_v, out_hbm.at[pl.ds(off, BATCH)])
        return carry

    lax.fori_loop(0, NB, body, 0)


_gather = pl.kernel(
    _gather_body,
    out_type=jax.ShapeDtypeStruct((E, D), jnp.float32),
    mesh=_sc_mesh,
    scratch_types=[
        pltpu.VMEM((BATCH,), jnp.int32),
        pltpu.VMEM((BATCH, D), jnp.float32),
        pltpu.SemaphoreType.DMA,
    ],
)


def _scatter_body(m_hbm, dst_hbm, zeros_hbm, out_hbm, idx_v, rows_v, acc, sem):
    c = lax.axis_index("c")
    s = lax.axis_index("s")
    pltpu.sync_copy(zeros_hbm.at[pl.ds(s * RPS, RPS)], acc.at[pl.ds(s * RPS, RPS)])
    plsc.subcore_barrier()

    base = (s * NC + c) * EPW

    def body(i, carry):
        off = base + i * BATCH
        pltpu.sync_copy(dst_hbm.at[pl.ds(off, BATCH)], idx_v)
        pltpu.sync_copy(m_hbm.at[pl.ds(off, BATCH)], rows_v)
        pltpu.sync_copy(rows_v, acc.at[idx_v], add=True)
        return carry

    lax.fori_loop(0, NB, body, 0)
    plsc.subcore_barrier()
    pltpu.sync_copy(acc.at[pl.ds(s * RPS, RPS)],
                    out_hbm.at[c, pl.ds(s * RPS, RPS)])


_scatter = pl.kernel(
    _scatter_body,
    out_type=jax.ShapeDtypeStruct((NC, NP, D), jnp.float32),
    mesh=_sc_mesh,
    scratch_types=[
        pltpu.VMEM((BATCH,), jnp.int32),
        pltpu.VMEM((BATCH, D), jnp.float32),
        pltpu.VMEM_SHARED((NP, D), jnp.float32),
        pltpu.SemaphoreType.DMA,
    ],
)


def _a1_body(h_ref, w_ref, b_ref, o_ref):
    o_ref[...] = (
        jnp.dot(h_ref[...], w_ref[...], preferred_element_type=jnp.float32)
        + b_ref[...]
    )


def _mlp_body(aj_ref, ea_ref, w1b_ref, w2_ref, b2_ref, o_ref):
    x = aj_ref[...] + jnp.dot(
        ea_ref[...], w1b_ref[...], preferred_element_type=jnp.float32
    )
    h = 0.5 * x * (1.0 + lax.erf(x * 0.7071067811865476))
    o_ref[...] = (
        jnp.dot(h, w2_ref[...], preferred_element_type=jnp.float32) + b2_ref[...]
    )


def _ln_body(h_ref, agg_ref, g_ref, beta_ref, o_ref):
    x = h_ref[...] + agg_ref[0] + agg_ref[1]
    mu = jnp.mean(x, axis=-1, keepdims=True)
    xc = x - mu
    var = jnp.mean(xc * xc, axis=-1, keepdims=True)
    o_ref[...] = xc * lax.rsqrt(var + 1e-5) * g_ref[...] + beta_ref[...]


_NBLK = 1000
_EBLK = 4000


def kernel(H, edge_index, edge_attr, W1, b1, W2, b2, gamma, beta):
    src = edge_index[0].astype(jnp.int32)
    dst = edge_index[1].astype(jnp.int32)
    W1a = W1[:D]
    W1b = W1[D:]
    b1r = b1.reshape(1, D)
    b2r = b2.reshape(1, D)
    gr = gamma.reshape(1, D)
    br = beta.reshape(1, D)

    A1 = pl.pallas_call(
        _a1_body,
        grid=(N // _NBLK,),
        in_specs=[
            pl.BlockSpec((_NBLK, D), lambda i: (i, 0)),
            pl.BlockSpec((D, D), lambda i: (0, 0)),
            pl.BlockSpec((1, D), lambda i: (0, 0)),
        ],
        out_specs=pl.BlockSpec((_NBLK, D), lambda i: (i, 0)),
        out_shape=jax.ShapeDtypeStruct((N, D), jnp.float32),
    )(H, W1a, b1r)

    Aj = _gather(A1, src)

    M = pl.pallas_call(
        _mlp_body,
        grid=(E // _EBLK,),
        in_specs=[
            pl.BlockSpec((_EBLK, D), lambda i: (i, 0)),
            pl.BlockSpec((_EBLK, DE), lambda i: (i, 0)),
            pl.BlockSpec((DE, D), lambda i: (0, 0)),
            pl.BlockSpec((D, D), lambda i: (0, 0)),
            pl.BlockSpec((1, D), lambda i: (0, 0)),
        ],
        out_specs=pl.BlockSpec((_EBLK, D), lambda i: (i, 0)),
        out_shape=jax.ShapeDtypeStruct((E, D), jnp.float32),
    )(Aj, edge_attr, W1b, W2, b2r)

    agg = _scatter(M, dst, jnp.zeros((NP, D), jnp.float32))

    out = pl.pallas_call(
        _ln_body,
        grid=(N // _NBLK,),
        in_specs=[
            pl.BlockSpec((_NBLK, D), lambda i: (i, 0)),
            pl.BlockSpec((NC, _NBLK, D), lambda i: (0, i, 0)),
            pl.BlockSpec((1, D), lambda i: (0, 0)),
            pl.BlockSpec((1, D), lambda i: (0, 0)),
        ],
        out_specs=pl.BlockSpec((_NBLK, D), lambda i: (i, 0)),
        out_shape=jax.ShapeDtypeStruct((N, D), jnp.float32),
    )(H, agg, gr, br)

    return out


# speedup vs baseline: 4.8038x; 1.5363x over previous
import functools

import jax
import jax.numpy as jnp
from jax import lax
from jax.experimental import pallas as pl
from jax.experimental.pallas import tpu as pltpu
from jax.experimental.pallas import tpu_sc as plsc

N = 10000
E = 320000
D = 128
DE = 16

NC = 2
NS = 16
NW = NC * NS
EPW = E // NW
BATCH = 80
NB = EPW // BATCH
NBUF = 5
SBATCH = 40
SNB = EPW // SBATCH
NP = 10240
RPS = NP // NS

_sc_mesh = plsc.VectorSubcoreMesh(core_axis_name="c", subcore_axis_name="s")


def _gather_body(table_hbm, idx_hbm, out_hbm, idx_all, rows, sems):
    c = lax.axis_index("c")
    s = lax.axis_index("s")
    wid = s * NC + c
    base = wid * EPW
    pltpu.sync_copy(idx_hbm.at[wid], idx_all)

    def gcopy(j, b):
        return pltpu.make_async_copy(
            table_hbm.at[idx_all.at[j]], rows.at[b], sems.at[b]
        )

    for b in range(NBUF):
        gcopy(b, b).start()

    @pl.loop(0, NB, step=NBUF)
    def _outer(i):
        for k in range(NBUF):
            j = i + k
            gcopy(j, k).wait()
            pltpu.sync_copy(rows.at[k], out_hbm.at[pl.ds(base + j * BATCH, BATCH)])
            nj = j + NBUF

            @pl.when(nj < NB)
            def _():
                gcopy(nj, k).start()


_gather = pl.kernel(
    _gather_body,
    out_type=jax.ShapeDtypeStruct((E, D), jnp.float32),
    mesh=_sc_mesh,
    scratch_types=[
        pltpu.VMEM((NB, BATCH), jnp.int32),
        pltpu.VMEM((NBUF, BATCH, D), jnp.float32),
        pltpu.SemaphoreType.DMA((NBUF,)),
    ],
)


def _scatter_body(m_hbm, dst_hbm, zeros_hbm, out_hbm, ibuf, rows, acc, msems, isems):
    c = lax.axis_index("c")
    s = lax.axis_index("s")
    wid = s * NC + c
    base = wid * EPW
    pltpu.sync_copy(zeros_hbm, acc.at[pl.ds(s * RPS, RPS)])
    plsc.subcore_barrier()

    def mcopy(j, b):
        return pltpu.make_async_copy(
            m_hbm.at[pl.ds(base + j * SBATCH, SBATCH)], rows.at[b], msems.at[b]
        )

    def icopy(j, b):
        return pltpu.make_async_copy(dst_hbm.at[wid, j], ibuf.at[b], isems.at[b])

    for b in range(NBUF):
        mcopy(b, b).start()
        icopy(b, b).start()

    @pl.loop(0, SNB, step=NBUF)
    def _outer(i):
        for k in range(NBUF):
            j = i + k
            mcopy(j, k).wait()
            icopy(j, k).wait()
            pltpu.sync_copy(rows.at[k], acc.at[ibuf.at[k]], add=True)
            nj = j + NBUF

            @pl.when(nj < SNB)
            def _():
                mcopy(nj, k).start()
                icopy(nj, k).start()

    plsc.subcore_barrier()
    pltpu.sync_copy(acc.at[pl.ds(s * RPS, RPS)],
                    out_hbm.at[c, pl.ds(s * RPS, RPS)])


_scatter = pl.kernel(
    _scatter_body,
    out_type=jax.ShapeDtypeStruct((NC, NP, D), jnp.float32),
    mesh=_sc_mesh,
    scratch_types=[
        pltpu.VMEM((NBUF, SBATCH), jnp.int32),
        pltpu.VMEM((NBUF, SBATCH, D), jnp.float32),
        pltpu.VMEM_SHARED((NP, D), jnp.float32),
        pltpu.SemaphoreType.DMA((NBUF,)),
        pltpu.SemaphoreType.DMA((NBUF,)),
    ],
)


def _a1_body(h_ref, w_ref, b_ref, o_ref):
    o_ref[...] = (
        jnp.dot(h_ref[...], w_ref[...], preferred_element_type=jnp.float32)
        + b_ref[...]
    )


def _mlp_body(aj_ref, ea_ref, w1b_ref, w2_ref, b2_ref, o_ref):
    x = aj_ref[...] + jnp.dot(
        ea_ref[...], w1b_ref[...], preferred_element_type=jnp.float32
    )
    h = 0.5 * x * (1.0 + lax.erf(x * 0.7071067811865476))
    o_ref[...] = (
        jnp.dot(h, w2_ref[...], preferred_element_type=jnp.float32) + b2_ref[...]
    )


def _ln_body(h_ref, agg_ref, g_ref, beta_ref, o_ref):
    x = h_ref[...] + agg_ref[0] + agg_ref[1]
    mu = jnp.mean(x, axis=-1, keepdims=True)
    xc = x - mu
    var = jnp.mean(xc * xc, axis=-1, keepdims=True)
    o_ref[...] = xc * lax.rsqrt(var + 1e-5) * g_ref[...] + beta_ref[...]


_NBLK = 1000
_EBLK = 4000


def kernel(H, edge_index, edge_attr, W1, b1, W2, b2, gamma, beta):
    src = edge_index[0].astype(jnp.int32).reshape(NW, NB, BATCH)
    dst = edge_index[1].astype(jnp.int32).reshape(NW, SNB, SBATCH)
    W1a = W1[:D]
    W1b = W1[D:]
    b1r = b1.reshape(1, D)
    b2r = b2.reshape(1, D)
    gr = gamma.reshape(1, D)
    br = beta.reshape(1, D)

    A1 = pl.pallas_call(
        _a1_body,
        grid=(N // _NBLK,),
        in_specs=[
            pl.BlockSpec((_NBLK, D), lambda i: (i, 0)),
            pl.BlockSpec((D, D), lambda i: (0, 0)),
            pl.BlockSpec((1, D), lambda i: (0, 0)),
        ],
        out_specs=pl.BlockSpec((_NBLK, D), lambda i: (i, 0)),
        out_shape=jax.ShapeDtypeStruct((N, D), jnp.float32),
    )(H, W1a, b1r)

    Aj = _gather(A1, src)

    M = pl.pallas_call(
        _mlp_body,
        grid=(E // _EBLK,),
        in_specs=[
            pl.BlockSpec((_EBLK, D), lambda i: (i, 0)),
            pl.BlockSpec((_EBLK, DE), lambda i: (i, 0)),
            pl.BlockSpec((DE, D), lambda i: (0, 0)),
            pl.BlockSpec((D, D), lambda i: (0, 0)),
            pl.BlockSpec((1, D), lambda i: (0, 0)),
        ],
        out_specs=pl.BlockSpec((_EBLK, D), lambda i: (i, 0)),
        out_shape=jax.ShapeDtypeStruct((E, D), jnp.float32),
    )(Aj, edge_attr, W1b, W2, b2r)

    agg = _scatter(M, dst, jnp.zeros((RPS, D), jnp.float32))

    out = pl.pallas_call(
        _ln_body,
        grid=(N // _NBLK,),
        in_specs=[
            pl.BlockSpec((_NBLK, D), lambda i: (i, 0)),
            pl.BlockSpec((NC, _NBLK, D), lambda i: (0, i, 0)),
            pl.BlockSpec((1, D), lambda i: (0, 0)),
            pl.BlockSpec((1, D), lambda i: (0, 0)),
        ],
        out_specs=pl.BlockSpec((_NBLK, D), lambda i: (i, 0)),
        out_shape=jax.ShapeDtypeStruct((N, D), jnp.float32),
    )(H, agg, gr, br)

    return out


# speedup vs baseline: 5.4143x; 1.1271x over previous
import functools

import jax
import jax.numpy as jnp
from jax import lax
from jax.experimental import pallas as pl
from jax.experimental.pallas import tpu as pltpu
from jax.experimental.pallas import tpu_sc as plsc

N = 10000
E = 320000
D = 128
DE = 16

NC = 2
NS = 16
NW = NC * NS
EPW = E // NW
BATCH = 80
NB = EPW // BATCH
NBUF = 5
SBATCH = 40
SNB = EPW // SBATCH
NP = 10240
RPS = NP // NS

_sc_mesh = plsc.VectorSubcoreMesh(core_axis_name="c", subcore_axis_name="s")


def _gather_body(table_hbm, idx_hbm, out_hbm, ibuf, rows, table_s, wsems, isems):
    c = lax.axis_index("c")
    s = lax.axis_index("s")
    wid = s * NC + c
    base = wid * EPW

    def icopy(j, b):
        return pltpu.make_async_copy(idx_hbm.at[wid, j], ibuf.at[b], isems.at[b])

    for b in range(NBUF):
        icopy(b, b).start()
    @pl.when(s < NS - 1)
    def _():
        pltpu.sync_copy(table_hbm.at[pl.ds(s * 640, 640)],
                        table_s.at[pl.ds(s * 640, 640)])

    @pl.when(s == NS - 1)
    def _():
        pltpu.sync_copy(table_hbm.at[pl.ds(9600, N - 9600)],
                        table_s.at[pl.ds(9600, N - 9600)])

    plsc.subcore_barrier()

    def wcopy(j, b):
        return pltpu.make_async_copy(
            rows.at[b], out_hbm.at[pl.ds(base + j * SBATCH, SBATCH)], wsems.at[b]
        )

    @pl.loop(0, SNB, step=NBUF)
    def _outer(i):
        for k in range(NBUF):
            j = i + k

            @pl.when(j >= NBUF)
            def _():
                wcopy(j - NBUF, k).wait()

            icopy(j, k).wait()
            pltpu.sync_copy(table_s.at[ibuf.at[k]], rows.at[k])
            nj = j + NBUF

            @pl.when(nj < SNB)
            def _():
                icopy(nj, k).start()

            wcopy(j, k).start()

    for k in range(NBUF):
        wcopy(SNB - NBUF + k, k).wait()


_gather = pl.kernel(
    _gather_body,
    out_type=jax.ShapeDtypeStruct((E, D), jnp.float32),
    mesh=_sc_mesh,
    scratch_types=[
        pltpu.VMEM((NBUF, SBATCH), jnp.int32),
        pltpu.VMEM((NBUF, SBATCH, D), jnp.float32),
        pltpu.VMEM_SHARED((N, D), jnp.float32),
        pltpu.SemaphoreType.DMA((NBUF,)),
        pltpu.SemaphoreType.DMA((NBUF,)),
    ],
)


def _scatter_body(m_hbm, dst_hbm, zeros_hbm, out_hbm, ibuf, rows, acc, msems, isems):
    c = lax.axis_index("c")
    s = lax.axis_index("s")
    wid = s * NC + c
    base = wid * EPW
    pltpu.sync_copy(zeros_hbm, acc.at[pl.ds(s * RPS, RPS)])
    plsc.subcore_barrier()

    def mcopy(j, b):
        return pltpu.make_async_copy(
            m_hbm.at[pl.ds(base + j * SBATCH, SBATCH)], rows.at[b], msems.at[b]
        )

    def icopy(j, b):
        return pltpu.make_async_copy(dst_hbm.at[wid, j], ibuf.at[b], isems.at[b])

    for b in range(NBUF):
        mcopy(b, b).start()
        icopy(b, b).start()

    @pl.loop(0, SNB, step=NBUF)
    def _outer(i):
        for k in range(NBUF):
            j = i + k
            mcopy(j, k).wait()
            icopy(j, k).wait()
            pltpu.sync_copy(rows.at[k], acc.at[ibuf.at[k]], add=True)
            nj = j + NBUF

            @pl.when(nj < SNB)
            def _():
                mcopy(nj, k).start()
                icopy(nj, k).start()

    plsc.subcore_barrier()
    pltpu.sync_copy(acc.at[pl.ds(s * RPS, RPS)],
                    out_hbm.at[c, pl.ds(s * RPS, RPS)])


_scatter = pl.kernel(
    _scatter_body,
    out_type=jax.ShapeDtypeStruct((NC, NP, D), jnp.float32),
    mesh=_sc_mesh,
    scratch_types=[
        pltpu.VMEM((NBUF, SBATCH), jnp.int32),
        pltpu.VMEM((NBUF, SBATCH, D), jnp.float32),
        pltpu.VMEM_SHARED((NP, D), jnp.float32),
        pltpu.SemaphoreType.DMA((NBUF,)),
        pltpu.SemaphoreType.DMA((NBUF,)),
    ],
)


def _a1_body(h_ref, w_ref, b_ref, o_ref):
    o_ref[...] = (
        jnp.dot(h_ref[...], w_ref[...], preferred_element_type=jnp.float32)
        + b_ref[...]
    )


def _mlp_body(aj_ref, ea_ref, w1b_ref, w2_ref, b2_ref, o_ref):
    x = aj_ref[...] + jnp.dot(
        ea_ref[...], w1b_ref[...], preferred_element_type=jnp.float32
    )
    h = 0.5 * x * (1.0 + lax.erf(x * 0.7071067811865476))
    o_ref[...] = (
        jnp.dot(h.astype(jnp.bfloat16), w2_ref[...],
                preferred_element_type=jnp.float32)
        + b2_ref[...]
    )


def _ln_body(h_ref, agg_ref, g_ref, beta_ref, o_ref):
    x = h_ref[...] + agg_ref[0] + agg_ref[1]
    mu = jnp.mean(x, axis=-1, keepdims=True)
    xc = x - mu
    var = jnp.mean(xc * xc, axis=-1, keepdims=True)
    o_ref[...] = xc * lax.rsqrt(var + 1e-5) * g_ref[...] + beta_ref[...]


_NBLK = 1000
_EBLK = 4000


def kernel(H, edge_index, edge_attr, W1, b1, W2, b2, gamma, beta):
    src = edge_index[0].astype(jnp.int32).reshape(NW, SNB, SBATCH)
    dst = edge_index[1].astype(jnp.int32).reshape(NW, SNB, SBATCH)
    W1a = W1[:D]
    W1b = W1[D:]
    b1r = b1.reshape(1, D)
    b2r = b2.reshape(1, D)
    gr = gamma.reshape(1, D)
    br = beta.reshape(1, D)

    A1 = pl.pallas_call(
        _a1_body,
        grid=(N // _NBLK,),
        in_specs=[
            pl.BlockSpec((_NBLK, D), lambda i: (i, 0)),
            pl.BlockSpec((D, D), lambda i: (0, 0)),
            pl.BlockSpec((1, D), lambda i: (0, 0)),
        ],
        out_specs=pl.BlockSpec((_NBLK, D), lambda i: (i, 0)),
        out_shape=jax.ShapeDtypeStruct((N, D), jnp.float32),
    )(H, W1a, b1r)

    Aj = _gather(A1, src)

    M = pl.pallas_call(
        _mlp_body,
        grid=(E // _EBLK,),
        in_specs=[
            pl.BlockSpec((_EBLK, D), lambda i: (i, 0)),
            pl.BlockSpec((_EBLK, DE), lambda i: (i, 0)),
            pl.BlockSpec((DE, D), lambda i: (0, 0)),
            pl.BlockSpec((D, D), lambda i: (0, 0)),
            pl.BlockSpec((1, D), lambda i: (0, 0)),
        ],
        out_specs=pl.BlockSpec((_EBLK, D), lambda i: (i, 0)),
        out_shape=jax.ShapeDtypeStruct((E, D), jnp.float32),
    )(Aj, edge_attr, W1b, W2.astype(jnp.bfloat16), b2r)

    agg = _scatter(M, dst, jnp.zeros((RPS, D), jnp.float32))

    out = pl.pallas_call(
        _ln_body,
        grid=(N // _NBLK,),
        in_specs=[
            pl.BlockSpec((_NBLK, D), lambda i: (i, 0)),
            pl.BlockSpec((NC, _NBLK, D), lambda i: (0, i, 0)),
            pl.BlockSpec((1, D), lambda i: (0, 0)),
            pl.BlockSpec((1, D), lambda i: (0, 0)),
        ],
        out_specs=pl.BlockSpec((_NBLK, D), lambda i: (i, 0)),
        out_shape=jax.ShapeDtypeStruct((N, D), jnp.float32),
    )(H, agg, gr, br)

    return out
